# baseline (speedup 1.0000x reference)
import dataclasses
import functools

import jax
import jax.numpy as jnp
import numpy as np
from jax import lax
from jax.experimental import pallas as pl
from jax.experimental.pallas import tpu as pltpu
from jax.experimental.pallas import tpu_sc as plsc

N = 10000
NP = 10240
E = 320000
D = 128
NC = 2
NS = 16
NW = NC * NS
EPT = E // NW
K = 80
NBUF = 3
NCH = EPT // K
KD = 10000
STRIPE = NP // NS
ZROWS = 16

_mesh = plsc.VectorSubcoreMesh(core_axis_name="core", subcore_axis_name="subcore")

_cp = pltpu.CompilerParams()
if "needs_layout_passes" in pltpu.CompilerParams.__dataclass_fields__:
  _cp = dataclasses.replace(_cp, needs_layout_passes=False)
if "use_tc_tiling_on_sc" in pltpu.CompilerParams.__dataclass_fields__:
  _cp = dataclasses.replace(_cp, use_tc_tiling_on_sc=False)


def _zero_rows(zb, width):
  @pl.loop(0, ZROWS)
  def _(i):
    @pl.loop(0, width // 16)
    def _(j):
      zb[i, pl.ds(j * 16, 16)] = jnp.zeros((16,), jnp.float32)


@functools.partial(
    pl.kernel,
    out_type=jax.ShapeDtypeStruct((NC, 2, NP), jnp.float32),
    mesh=_mesh,
    scratch_types=[
        pltpu.VMEM_SHARED((NP,), jnp.float32),
        pltpu.VMEM_SHARED((NP,), jnp.float32),
        pltpu.VMEM((KD,), jnp.float32),
        pltpu.VMEM((KD,), jnp.int32),
        pltpu.VMEM((STRIPE,), jnp.float32),
    ],
)
def _degrees(src_hbm, dst_hbm, out_hbm, deg_s, deg_d, ones_v, idx_v, zb):
  cid = lax.axis_index("core")
  sid = lax.axis_index("subcore")
  wid = cid * NS + sid

  @pl.loop(0, STRIPE // 16)
  def _(i):
    zb[pl.ds(i * 16, 16)] = jnp.zeros((16,), jnp.float32)

  @pl.loop(0, KD // 16)
  def _(i):
    ones_v[pl.ds(i * 16, 16)] = jnp.full((16,), 1.0, jnp.float32)

  sl = pl.ds(sid * STRIPE, STRIPE)
  pltpu.sync_copy(zb, deg_s.at[sl])
  pltpu.sync_copy(zb, deg_d.at[sl])

  plsc.subcore_barrier()

  @pl.loop(0, EPT // KD)
  def _(c):
    base = wid * EPT + c * KD
    pltpu.sync_copy(src_hbm.at[pl.ds(base, KD)], idx_v)
    pltpu.sync_copy(ones_v, deg_s.at[idx_v], add=True)
    pltpu.sync_copy(dst_hbm.at[pl.ds(base, KD)], idx_v)
    pltpu.sync_copy(ones_v, deg_d.at[idx_v], add=True)

  plsc.subcore_barrier()

  pltpu.sync_copy(deg_s.at[sl], out_hbm.at[cid, 0, sl])
  pltpu.sync_copy(deg_d.at[sl], out_hbm.at[cid, 1, sl])


@functools.partial(
    pl.kernel,
    out_type=jax.ShapeDtypeStruct((NC, NP, D), jnp.float32),
    mesh=_mesh,
    compiler_params=_cp,
    scratch_types=[
        pltpu.VMEM_SHARED((NP, D), jnp.float32),
        [pltpu.VMEM((K, D // 2), jnp.int32)] * NBUF,
        [pltpu.VMEM((K, D), jnp.float32)] * NBUF,
        [pltpu.VMEM((K,), jnp.int32)] * NBUF,
        [pltpu.VMEM((K,), jnp.int32)] * NBUF,
        pltpu.VMEM((ZROWS, D), jnp.float32),
        [pltpu.SemaphoreType.DMA] * NBUF,
        [pltpu.SemaphoreType.DMA] * NBUF,
    ],
)
def _propagate(xsp_hbm, src_hbm, dst_hbm, out_hbm, acc, rows16, rows32,
               sidx, didx, zb, gsem, ssem):
  cid = lax.axis_index("core")
  sid = lax.axis_index("subcore")
  wid = cid * NS + sid

  _zero_rows(zb, D)

  @pl.loop(0, STRIPE // ZROWS)
  def _(r):
    pltpu.sync_copy(zb, acc.at[pl.ds(sid * STRIPE + r * ZROWS, ZROWS)])

  plsc.subcore_barrier()

  def load_idx_and_gather(c, b):
    base = wid * EPT + c * K
    pltpu.sync_copy(src_hbm.at[pl.ds(base, K)], sidx[b])
    pltpu.sync_copy(dst_hbm.at[pl.ds(base, K)], didx[b])
    pltpu.async_copy(xsp_hbm.at[sidx[b]], rows16[b], gsem[b])

  def wait_gather(b):
    pltpu.make_async_copy(xsp_hbm.at[pl.ds(0, K)], rows16[b], gsem[b]).wait()

  def wait_scatter(b):
    pltpu.make_async_copy(rows32[b], acc.at[pl.ds(0, K)], ssem[b]).wait()

  def unpack_rows(b):
    @pl.loop(0, K // 4)
    def _(r4):
      for rr in range(4):
        for q in range(D // 32):
          v = rows16[b][r4 * 4 + rr, pl.ds(q * 16, 16)]
          vbf = plsc.bitcast(v, jnp.bfloat16)
          lo, hi = plsc.unpack(vbf, format=plsc.PackFormat.INTERLEAVED)
          rows32[b][r4 * 4 + rr, pl.ds(q * 32, 16)] = lo
          rows32[b][r4 * 4 + rr, pl.ds(q * 32 + 16, 16)] = hi

  def scatter(b):
    unpack_rows(b)
    pltpu.async_copy(rows32[b], acc.at[didx[b]], ssem[b], add=True)

  for b in range(NBUF):
    load_idx_and_gather(b, b)

  @pl.loop(0, (NCH - 2) // NBUF)
  def _(i):
    for b in range(NBUF):
      c = i * NBUF + b
      wait_gather(b)
      scatter(b)

      @pl.when(c + NBUF < NCH)
      def _():
        wait_scatter(b)
        load_idx_and_gather(c + NBUF, b)

  wait_gather(0)
  scatter(0)
  wait_gather(1)
  scatter(1)
  for b in range(NBUF):
    wait_scatter(b)

  plsc.subcore_barrier()

  sl = pl.ds(sid * STRIPE, STRIPE)
  pltpu.sync_copy(acc.at[sl], out_hbm.at[cid, sl])


def _norm_cols(degp, col):
  deg = degp[0, col] + degp[1, col]
  ns = jnp.where(deg > 0, lax.rsqrt(deg), 0.0)
  return ns[:, None]


def _prep_body(x_ref, degp_ref, xs_ref):
  ns = _norm_cols(degp_ref[...], 0)
  xs_ref[...] = x_ref[...] * ns[:N]


_prep = pl.pallas_call(
    _prep_body, out_shape=jax.ShapeDtypeStruct((N, D), jnp.float32))


def _mid_body(accp_ref, degp_ref, w_ref, b_ref, o_ref):
  degp = degp_ref[...]
  nd = _norm_cols(degp, 1)
  ns = _norm_cols(degp, 0)
  agg = (accp_ref[0, :N] + accp_ref[1, :N]) * nd[:N]
  h = jnp.dot(agg, w_ref[...], preferred_element_type=jnp.float32) + b_ref[...]
  h = jnp.maximum(h, 0.0)
  o_ref[...] = h * ns[:N]


_mid = pl.pallas_call(
    _mid_body, out_shape=jax.ShapeDtypeStruct((N, D), jnp.float32))


def _fin_body(accp_ref, degp_ref, w_ref, b_ref, o_ref):
  nd = _norm_cols(degp_ref[...], 1)
  agg = (accp_ref[0, :N] + accp_ref[1, :N]) * nd[:N]
  o_ref[...] = (
      jnp.dot(agg, w_ref[...], preferred_element_type=jnp.float32) + b_ref[...])


_fin = pl.pallas_call(
    _fin_body, out_shape=jax.ShapeDtypeStruct((N, D), jnp.float32))


_PERM = jnp.asarray(np.concatenate([
    32 * q + np.concatenate([np.arange(0, 32, 2), np.arange(1, 32, 2)])
    for q in range(D // 32)]).astype(np.int32))


def _pack(m):
  mb = m.astype(jnp.bfloat16).reshape(N, D // 2, 2)
  return jax.lax.bitcast_convert_type(mb, jnp.int32)


@jax.jit
def kernel(x, edge_index, W1, b1, W2, b2):
  src = edge_index[0]
  dst = edge_index[1]
  degp = _degrees(src, dst)
  xs1 = _prep(x, degp)
  accp1 = _propagate(_pack(xs1), src, dst)
  xs2 = _mid(accp1, degp, W1[_PERM, :], b1)
  accp2 = _propagate(_pack(xs2), src, dst)
  return _fin(accp2, degp, W2[_PERM, :], b2)


# speedup vs baseline: 1.9968x; 1.9968x over previous
import functools

import jax
import jax.numpy as jnp
from jax import lax
from jax.experimental import pallas as pl
from jax.experimental.pallas import tpu as pltpu
from jax.experimental.pallas import tpu_sc as plsc

N = 10000
NP = 10240
E = 320000
D = 128
NC = 2
NS = 16
NW = NC * NS
EPT = E // NW
K = 80
NBUF = 4
NCH = EPT // K
KD = 10000
STRIPE = NP // NS
ZROWS = 32

_mesh = plsc.VectorSubcoreMesh(core_axis_name="core", subcore_axis_name="subcore")


def _zero_rows(zb, width):
  @pl.loop(0, ZROWS)
  def _(i):
    @pl.loop(0, width // 16)
    def _(j):
      zb[i, pl.ds(j * 16, 16)] = jnp.zeros((16,), jnp.float32)


@functools.partial(
    pl.kernel,
    out_type=jax.ShapeDtypeStruct((NC, 2, NP), jnp.float32),
    mesh=_mesh,
    scratch_types=[
        pltpu.VMEM_SHARED((NP,), jnp.float32),
        pltpu.VMEM_SHARED((NP,), jnp.float32),
        pltpu.VMEM((KD,), jnp.float32),
        pltpu.VMEM((KD,), jnp.int32),
        pltpu.VMEM((STRIPE,), jnp.float32),
    ],
)
def _degrees(src_hbm, dst_hbm, out_hbm, deg_s, deg_d, ones_v, idx_v, zb):
  cid = lax.axis_index("core")
  sid = lax.axis_index("subcore")
  wid = cid * NS + sid

  @pl.loop(0, STRIPE // 16)
  def _(i):
    zb[pl.ds(i * 16, 16)] = jnp.zeros((16,), jnp.float32)

  @pl.loop(0, KD // 16)
  def _(i):
    ones_v[pl.ds(i * 16, 16)] = jnp.full((16,), 1.0, jnp.float32)

  sl = pl.ds(sid * STRIPE, STRIPE)
  pltpu.sync_copy(zb, deg_s.at[sl])
  pltpu.sync_copy(zb, deg_d.at[sl])

  plsc.subcore_barrier()

  @pl.loop(0, EPT // KD)
  def _(c):
    base = wid * EPT + c * KD
    pltpu.sync_copy(src_hbm.at[pl.ds(base, KD)], idx_v)
    pltpu.sync_copy(ones_v, deg_s.at[idx_v], add=True)
    pltpu.sync_copy(dst_hbm.at[pl.ds(base, KD)], idx_v)
    pltpu.sync_copy(ones_v, deg_d.at[idx_v], add=True)

  plsc.subcore_barrier()

  pltpu.sync_copy(deg_s.at[sl], out_hbm.at[cid, 0, sl])
  pltpu.sync_copy(deg_d.at[sl], out_hbm.at[cid, 1, sl])


@functools.partial(
    pl.kernel,
    out_type=jax.ShapeDtypeStruct((NC, NP, D), jnp.float32),
    mesh=_mesh,
    scratch_types=[
        pltpu.VMEM_SHARED((NP, D), jnp.float32),
        [pltpu.VMEM((K, D), jnp.float32)] * NBUF,
        [pltpu.VMEM((K,), jnp.int32)] * NBUF,
        [pltpu.VMEM((K,), jnp.int32)] * NBUF,
        pltpu.VMEM((ZROWS, D), jnp.float32),
        [pltpu.SemaphoreType.DMA] * NBUF,
        [pltpu.SemaphoreType.DMA] * NBUF,
    ],
)
def _propagate(xs_hbm, src_hbm, dst_hbm, out_hbm, acc, rows, sidx, didx, zb,
               gsem, ssem):
  cid = lax.axis_index("core")
  sid = lax.axis_index("subcore")
  wid = cid * NS + sid

  _zero_rows(zb, D)

  @pl.loop(0, STRIPE // ZROWS)
  def _(r):
    pltpu.sync_copy(zb, acc.at[pl.ds(sid * STRIPE + r * ZROWS, ZROWS)])

  plsc.subcore_barrier()

  def load_idx_and_gather(c, b):
    base = wid * EPT + c * K
    pltpu.sync_copy(src_hbm.at[pl.ds(base, K)], sidx[b])
    pltpu.sync_copy(dst_hbm.at[pl.ds(base, K)], didx[b])
    pltpu.async_copy(xs_hbm.at[sidx[b]], rows[b], gsem[b])

  def wait_gather(b):
    pltpu.make_async_copy(xs_hbm.at[pl.ds(0, K)], rows[b], gsem[b]).wait()

  def wait_scatter(b):
    pltpu.make_async_copy(rows[b], acc.at[pl.ds(0, K)], ssem[b]).wait()

  for b in range(NBUF):
    load_idx_and_gather(b, b)

  @pl.loop(0, (NCH - 1) // NBUF)
  def _(i):
    for b in range(NBUF):
      c = i * NBUF + b
      wait_gather(b)
      pltpu.async_copy(rows[b], acc.at[didx[b]], ssem[b], add=True)

      @pl.when(c + NBUF < NCH)
      def _():
        wait_scatter(b)
        load_idx_and_gather(c + NBUF, b)

  wait_gather(0)
  pltpu.async_copy(rows[0], acc.at[didx[0]], ssem[0], add=True)
  for b in range(NBUF):
    wait_scatter(b)

  plsc.subcore_barrier()

  sl = pl.ds(sid * STRIPE, STRIPE)
  pltpu.sync_copy(acc.at[sl], out_hbm.at[cid, sl])


def _norm_cols(degp, col):
  deg = degp[0, col] + degp[1, col]
  ns = jnp.where(deg > 0, lax.rsqrt(deg), 0.0)
  return ns[:, None]


def _prep_body(x_ref, degp_ref, xs_ref):
  ns = _norm_cols(degp_ref[...], 0)
  xs_ref[...] = x_ref[...] * ns[:N]


_prep = pl.pallas_call(
    _prep_body, out_shape=jax.ShapeDtypeStruct((N, D), jnp.float32))


def _mid_body(accp_ref, degp_ref, w_ref, b_ref, o_ref):
  degp = degp_ref[...]
  nd = _norm_cols(degp, 1)
  ns = _norm_cols(degp, 0)
  agg = (accp_ref[0, :N] + accp_ref[1, :N]) * nd[:N]
  h = jnp.dot(agg, w_ref[...], preferred_element_type=jnp.float32) + b_ref[...]
  h = jnp.maximum(h, 0.0)
  o_ref[...] = h * ns[:N]


_mid = pl.pallas_call(
    _mid_body, out_shape=jax.ShapeDtypeStruct((N, D), jnp.float32))


def _fin_body(accp_ref, degp_ref, w_ref, b_ref, o_ref):
  nd = _norm_cols(degp_ref[...], 1)
  agg = (accp_ref[0, :N] + accp_ref[1, :N]) * nd[:N]
  o_ref[...] = (
      jnp.dot(agg, w_ref[...], preferred_element_type=jnp.float32) + b_ref[...])


_fin = pl.pallas_call(
    _fin_body, out_shape=jax.ShapeDtypeStruct((N, D), jnp.float32))


@jax.jit
def kernel(x, edge_index, W1, b1, W2, b2):
  src = edge_index[0]
  dst = edge_index[1]
  degp = _degrees(src, dst)
  xs1 = _prep(x, degp)
  accp1 = _propagate(xs1, src, dst)
  xs2 = _mid(accp1, degp, W1, b1)
  accp2 = _propagate(xs2, src, dst)
  return _fin(accp2, degp, W2, b2)
